# Initial kernel scaffold; baseline (speedup 1.0000x reference)
import jax
import jax.numpy as jnp
from jax.experimental import pallas as pl


def kernel(x_a, x_p, x_v, edge_index_a, edge_index_p, edge_index_v, x_node, num_node, edge_weight_a, edge_weight_p, edge_weight_v, W_agg_a, b_agg_a, W_agg_p, b_agg_p, W_agg_v, b_agg_v, u, W_lin, b_lin):
    raise NotImplementedError("write your pallas kernel here")


# speedup vs baseline: 3.3459x; 3.3459x over previous
import functools

import jax
import jax.numpy as jnp
from jax import lax
from jax.experimental import pallas as pl
from jax.experimental.pallas import tpu as pltpu
from jax.experimental.pallas import tpu_sc as plsc

N = 10000
E = 320000
D = 128
DA = D + 16
NTILES = 32
EPT = E // NTILES
CH = 80
NCH = EPT // CH
NACC = 10240
RPT = NACC // 16
RBLK = 1024


def _pre_body(xa, xp, xv, wa, wp, wv, ba, bp, bv, ha, hp, hv):
    ha[...] = jnp.maximum(jnp.dot(xa[...], wa[...].T,
                                  preferred_element_type=jnp.float32) + ba[...], 0.0)
    hp[...] = jnp.maximum(jnp.dot(xp[...], wp[...].T,
                                  preferred_element_type=jnp.float32) + bp[...], 0.0)
    hv[...] = jnp.maximum(jnp.dot(xv[...], wv[...].T,
                                  preferred_element_type=jnp.float32) + bv[...], 0.0)


def _pre(x_a, x_p, x_v, W_a, W_p, W_v, b_a, b_p, b_v):
    xspec = pl.BlockSpec((RBLK, D), lambda i: (i, 0))
    wspec = pl.BlockSpec((D, D), lambda i: (0, 0))
    bspec = pl.BlockSpec((1, D), lambda i: (0, 0))
    return pl.pallas_call(
        _pre_body,
        grid=(pl.cdiv(N, RBLK),),
        in_specs=[xspec, xspec, xspec, wspec, wspec, wspec, bspec, bspec, bspec],
        out_specs=[xspec, xspec, xspec],
        out_shape=[jax.ShapeDtypeStruct((N, D), jnp.float32)] * 3,
    )(x_a, x_p, x_v, W_a, W_p, W_v,
      b_a.reshape(1, D), b_p.reshape(1, D), b_v.reshape(1, D))


EPT = E // NTILES
NCH = EPT // CH


def _sc_body(ha, hp, hv, srca, srcp, srcv, tgta, tgtp, tgtv,
             wea, wep, wev, zeros_hbm, cntrows_hbm,
             outa, outp, outv, cnt_out,
             src_c, tgt_v, w_v, grows, cnt_src, acc, sem, sem2):
    c = lax.axis_index("c")
    s = lax.axis_index("s")
    wid = c * 16 + s
    row0 = pl.multiple_of(s * RPT, 8)

    def zero_acc():
        pltpu.sync_copy(zeros_hbm, acc.at[pl.ds(row0, RPT), :])

    def writeout(dst_hbm):
        pltpu.sync_copy(acc.at[pl.ds(row0, RPT), :],
                        dst_hbm.at[c, pl.ds(row0, RPT), :])

    def do_data(h_hbm, src3, tgt3, we3, out_hbm):
        zero_acc()
        pltpu.sync_copy(tgt3.at[wid], tgt_v)
        plsc.subcore_barrier()

        def chunk(k, _):
            cg = pltpu.async_copy(h_hbm.at[tgt_v.at[k]], grows, sem)
            cw = pltpu.async_copy(we3.at[wid, k], w_v, sem2)
            cs = pltpu.async_copy(src3.at[wid, k], src_c, sem2)
            cg.wait()
            cw.wait()
            cs.wait()

            def edge(e, _):
                ws = w_v[e, :]
                for j in range(D // 16):
                    grows[e, pl.ds(j * 16, 16)] = \
                        grows[e, pl.ds(j * 16, 16)] * ws
                return _
            lax.fori_loop(0, CH, edge, None)

            pltpu.sync_copy(grows, acc.at[src_c], add=True)
            return _
        lax.fori_loop(0, NCH, chunk, None)
        plsc.subcore_barrier()
        writeout(out_hbm)
        plsc.subcore_barrier()

    def do_counts():
        zero_acc()
        plsc.subcore_barrier()
        for r, src3 in enumerate((srca, srcp, srcv)):
            pltpu.sync_copy(cntrows_hbm.at[r], cnt_src)

            def chunk(k, _):
                pltpu.sync_copy(src3.at[wid, k], src_c)
                pltpu.sync_copy(cnt_src, acc.at[src_c], add=True)
                return _
            lax.fori_loop(0, NCH, chunk, None)
        plsc.subcore_barrier()
        writeout(cnt_out)
        plsc.subcore_barrier()

    do_data(ha, srca, tgta, wea, outa)
    do_data(hp, srcp, tgtp, wep, outp)
    do_data(hv, srcv, tgtv, wev, outv)
    do_counts()


def _sc_edge_phase(ha, hp, hv, ei_a, ei_p, ei_v, ew_a, ew_p, ew_v):
    zeros = jnp.zeros((RPT, D), jnp.float32)
    lane = jnp.arange(D, dtype=jnp.int32)
    cntrows = jnp.stack([
        jnp.broadcast_to((lane == r).astype(jnp.float32), (CH, D))
        for r in range(3)])
    mesh = plsc.VectorSubcoreMesh(core_axis_name="c", subcore_axis_name="s")
    f = pl.kernel(
        _sc_body,
        out_type=[jax.ShapeDtypeStruct((2, NACC, D), jnp.float32)] * 4,
        mesh=mesh,
        scratch_types=[
            pltpu.VMEM((CH,), jnp.int32),
            pltpu.VMEM((NCH, CH), jnp.int32),
            pltpu.VMEM((CH, 16), jnp.float32),
            pltpu.VMEM((CH, D), jnp.float32),
            pltpu.VMEM((CH, D), jnp.float32),
            pltpu.VMEM_SHARED((NACC, D), jnp.float32),
            pltpu.SemaphoreType.DMA,
            pltpu.SemaphoreType.DMA,
        ],
    )
    r3 = lambda a: a.astype(jnp.int32).reshape(NTILES, NCH, CH)
    r4 = lambda w: jnp.broadcast_to(
        w[:, None], (E, 16)).reshape(NTILES, NCH, CH, 16)
    return f(ha, hp, hv,
             r3(ei_a[0]), r3(ei_p[0]), r3(ei_v[0]),
             r3(ei_a[1]), r3(ei_p[1]), r3(ei_v[1]),
             r4(ew_a), r4(ew_p), r4(ew_v),
             zeros, cntrows)


def _post_body(pa, pp, pv, cc, xn, u, wl, bl, out):
    x = xn[...]
    deg = cc[...][0] + cc[...][1]

    def unpack(p, r):
        pv2 = p[...]
        return (pv2[0] + pv2[1]) / jnp.maximum(deg[:, r:r + 1], 1.0)

    aggr_a = unpack(pa, 0)
    aggr_p = unpack(pp, 1)
    aggr_v = unpack(pv, 2)

    uu = u[...]
    u1 = uu[:D, :]
    u2 = uu[D:, :]
    xu = jnp.dot(x, u2, preferred_element_type=jnp.float32)

    def score(aggr):
        z = jnp.dot(aggr, u1, preferred_element_type=jnp.float32) + xu
        return jnp.exp(jnp.where(z > 0, z, 0.01 * z))

    sa = score(aggr_a)
    sp = score(aggr_p)
    sv = score(aggr_v)
    inv = 1.0 / (sa + sp + sv)
    comb = (sa * aggr_a + sp * aggr_p + sv * aggr_v) * inv

    w = wl[...]
    w1 = w[:, :D]
    w2 = w[:, D:]
    pre = jnp.dot(x, w1.T, preferred_element_type=jnp.float32) \
        + jnp.dot(comb, w2.T, preferred_element_type=jnp.float32) + bl[...]
    pre = jnp.maximum(pre, 0.0)
    norm = jnp.sqrt(jnp.sum(pre * pre, axis=1, keepdims=True))
    out[...] = pre / jnp.maximum(norm, 1e-12)


def _post(pa, pp, pv, cc, x_node, u, W_lin, b_lin):
    pspec = pl.BlockSpec((2, RBLK, D), lambda i: (0, i, 0))
    xspec = pl.BlockSpec((RBLK, D), lambda i: (i, 0))
    return pl.pallas_call(
        _post_body,
        grid=(pl.cdiv(N, RBLK),),
        in_specs=[pspec, pspec, pspec, pspec, xspec,
                  pl.BlockSpec((2 * D, 1), lambda i: (0, 0)),
                  pl.BlockSpec((D, 2 * D), lambda i: (0, 0)),
                  pl.BlockSpec((1, D), lambda i: (0, 0))],
        out_specs=xspec,
        out_shape=jax.ShapeDtypeStruct((N, D), jnp.float32),
    )(pa, pp, pv, cc, x_node, u, W_lin, b_lin.reshape(1, D))


def kernel(x_a, x_p, x_v, edge_index_a, edge_index_p, edge_index_v, x_node,
           num_node, edge_weight_a, edge_weight_p, edge_weight_v,
           W_agg_a, b_agg_a, W_agg_p, b_agg_p, W_agg_v, b_agg_v,
           u, W_lin, b_lin):
    ha, hp, hv = _pre(x_a, x_p, x_v, W_agg_a, W_agg_p, W_agg_v,
                      b_agg_a, b_agg_p, b_agg_v)
    pa, pp, pv, cc = _sc_edge_phase(
        ha, hp, hv, edge_index_a, edge_index_p, edge_index_v,
        edge_weight_a, edge_weight_p, edge_weight_v)
    return _post(pa, pp, pv, cc, x_node, u, W_lin, b_lin)


# speedup vs baseline: 3.4239x; 1.0233x over previous
import functools

import jax
import jax.numpy as jnp
from jax import lax
from jax.experimental import pallas as pl
from jax.experimental.pallas import tpu as pltpu
from jax.experimental.pallas import tpu_sc as plsc

N = 10000
E = 320000
D = 128
DA = D + 16
NTILES = 32
EPT = E // NTILES
CH = 80
NCH = EPT // CH
NACC = 10240
RPT = NACC // 16
RBLK = 1024


def _pre_body(xa, xp, xv, wa, wp, wv, ba, bp, bv, ha, hp, hv):
    ha[...] = jnp.maximum(jnp.dot(xa[...], wa[...].T,
                                  preferred_element_type=jnp.float32) + ba[...], 0.0)
    hp[...] = jnp.maximum(jnp.dot(xp[...], wp[...].T,
                                  preferred_element_type=jnp.float32) + bp[...], 0.0)
    hv[...] = jnp.maximum(jnp.dot(xv[...], wv[...].T,
                                  preferred_element_type=jnp.float32) + bv[...], 0.0)


def _pre(x_a, x_p, x_v, W_a, W_p, W_v, b_a, b_p, b_v):
    xspec = pl.BlockSpec((RBLK, D), lambda i: (i, 0))
    wspec = pl.BlockSpec((D, D), lambda i: (0, 0))
    bspec = pl.BlockSpec((1, D), lambda i: (0, 0))
    return pl.pallas_call(
        _pre_body,
        grid=(pl.cdiv(N, RBLK),),
        in_specs=[xspec, xspec, xspec, wspec, wspec, wspec, bspec, bspec, bspec],
        out_specs=[xspec, xspec, xspec],
        out_shape=[jax.ShapeDtypeStruct((N, D), jnp.float32)] * 3,
    )(x_a, x_p, x_v, W_a, W_p, W_v,
      b_a.reshape(1, D), b_p.reshape(1, D), b_v.reshape(1, D))


EPT = E // NTILES
NCH = EPT // CH
K3 = (NCH - 2) // 3
assert 3 * K3 + 2 == NCH


def _sc_body(ha, hp, hv, srca, srcp, srcv, tgta, tgtp, tgtv,
             wea, wep, wev, zeros_hbm, cntrows_hbm,
             outa, outp, outv, cnt_out,
             src_c0, src_c1, src_c2, tgt_v, w_v0, w_v1, w_v2,
             grows0, grows1, grows2,
             semg0, semg1, semg2, semw0, semw1, semw2,
             semc0, semc1, semc2, sems0, sems1, sems2, acc):
    c = lax.axis_index("c")
    s = lax.axis_index("s")
    wid = c * 16 + s
    row0 = pl.multiple_of(s * RPT, 8)

    grows = (grows0, grows1, grows2)
    w_v = (w_v0, w_v1, w_v2)
    src_c = (src_c0, src_c1, src_c2)
    semg = (semg0, semg1, semg2)
    semw = (semw0, semw1, semw2)
    semc = (semc0, semc1, semc2)
    sems = (sems0, sems1, sems2)

    def zero_acc():
        pltpu.sync_copy(zeros_hbm, acc.at[pl.ds(row0, RPT), :])

    def writeout(dst_hbm):
        pltpu.sync_copy(acc.at[pl.ds(row0, RPT), :],
                        dst_hbm.at[c, pl.ds(row0, RPT), :])

    def do_data(h_hbm, src3, tgt3, we3, out_hbm):
        zero_acc()
        pltpu.sync_copy(tgt3.at[pl.ds(wid * EPT, EPT)], tgt_v)
        plsc.subcore_barrier()

        def g_start(k, b):
            pltpu.async_copy(h_hbm.at[tgt_v.at[pl.ds(k * CH, CH)]],
                             grows[b], semg[b])
            pltpu.async_copy(we3.at[pl.ds((wid * EPT + k * CH) * 16, CH * 16)],
                             w_v[b], semw[b])
            pltpu.async_copy(src3.at[pl.ds(wid * EPT + k * CH, CH)],
                             src_c[b], semc[b])

        def g_wait(k, b):
            pltpu.make_async_copy(h_hbm.at[tgt_v.at[pl.ds(k * CH, CH)]],
                                  grows[b], semg[b]).wait()
            pltpu.make_async_copy(
                we3.at[pl.ds((wid * EPT + k * CH) * 16, CH * 16)],
                w_v[b], semw[b]).wait()
            pltpu.make_async_copy(src3.at[pl.ds(wid * EPT + k * CH, CH)],
                                  src_c[b], semc[b]).wait()

        def s_start(b):
            pltpu.async_copy(grows[b], acc.at[src_c[b]], sems[b], add=True)

        def s_wait(b):
            pltpu.make_async_copy(grows[b], acc.at[src_c[b]], sems[b]).wait()

        def mult(b):
            gb, wb = grows[b], w_v[b]

            def edge(e, _):
                ws = wb[pl.ds(e * 16, 16)]
                for j in range(D // 16):
                    gb[e, pl.ds(j * 16, 16)] = gb[e, pl.ds(j * 16, 16)] * ws
                return _
            lax.fori_loop(0, CH, edge, None)

        g_start(0, 0)
        g_start(1, 1)

        def trio(k3, _):
            j0 = 3 * k3
            for i in range(3):
                bn = (i + 2) % 3
                if i == 0:
                    @pl.when(k3 > 0)
                    def _w():
                        s_wait(bn)
                else:
                    s_wait(bn)
                g_start(j0 + i + 2, bn)
                g_wait(j0 + i, i)
                mult(i)
                s_start(i)
            return _
        lax.fori_loop(0, K3, trio, None)

        s_wait(2)
        g_wait(NCH - 2, 0)
        mult(0)
        s_start(0)
        g_wait(NCH - 1, 1)
        mult(1)
        s_start(1)
        s_wait(0)
        s_wait(1)
        plsc.subcore_barrier()
        writeout(out_hbm)
        plsc.subcore_barrier()

    def do_counts():
        zero_acc()
        plsc.subcore_barrier()

        def c_start(src3, k, b):
            pltpu.async_copy(src3.at[pl.ds(wid * EPT + k * CH, CH)],
                             src_c[b], semc[b])

        def c_wait(src3, k, b):
            pltpu.make_async_copy(src3.at[pl.ds(wid * EPT + k * CH, CH)],
                                  src_c[b], semc[b]).wait()

        def s_start(b):
            pltpu.async_copy(grows0, acc.at[src_c[b]], sems[b], add=True)

        def s_wait(b):
            pltpu.make_async_copy(grows0, acc.at[src_c[b]], sems[b]).wait()

        for r, src3 in enumerate((srca, srcp, srcv)):
            pltpu.sync_copy(cntrows_hbm.at[r], grows0)
            c_start(src3, 0, 0)
            c_start(src3, 1, 1)

            def trio(k3, _):
                j0 = 3 * k3
                for i in range(3):
                    bn = (i + 2) % 3
                    if i == 0:
                        @pl.when(k3 > 0)
                        def _w():
                            s_wait(bn)
                    else:
                        s_wait(bn)
                    c_start(src3, j0 + i + 2, bn)
                    c_wait(src3, j0 + i, i)
                    s_start(i)
                return _
            lax.fori_loop(0, K3, trio, None)

            s_wait(2)
            c_wait(src3, NCH - 2, 0)
            s_start(0)
            c_wait(src3, NCH - 1, 1)
            s_start(1)
            s_wait(0)
            s_wait(1)
        plsc.subcore_barrier()
        writeout(cnt_out)
        plsc.subcore_barrier()

    do_data(ha, srca, tgta, wea, outa)
    do_data(hp, srcp, tgtp, wep, outp)
    do_data(hv, srcv, tgtv, wev, outv)
    do_counts()


def _sc_edge_phase(ha, hp, hv, ei_a, ei_p, ei_v, ew_a, ew_p, ew_v):
    zeros = jnp.zeros((RPT, D), jnp.float32)
    lane = jnp.arange(D, dtype=jnp.int32)
    cntrows = jnp.stack([
        jnp.broadcast_to((lane == r).astype(jnp.float32), (CH, D))
        for r in range(3)])
    mesh = plsc.VectorSubcoreMesh(core_axis_name="c", subcore_axis_name="s")
    f = pl.kernel(
        _sc_body,
        out_type=[jax.ShapeDtypeStruct((2, NACC, D), jnp.float32)] * 4,
        mesh=mesh,
        scratch_types=[
            pltpu.VMEM((CH,), jnp.int32),
            pltpu.VMEM((CH,), jnp.int32),
            pltpu.VMEM((CH,), jnp.int32),
            pltpu.VMEM((EPT,), jnp.int32),
            pltpu.VMEM((CH * 16,), jnp.float32),
            pltpu.VMEM((CH * 16,), jnp.float32),
            pltpu.VMEM((CH * 16,), jnp.float32),
            pltpu.VMEM((CH, D), jnp.float32),
            pltpu.VMEM((CH, D), jnp.float32),
            pltpu.VMEM((CH, D), jnp.float32),
        ] + [pltpu.SemaphoreType.DMA] * 12 + [
            pltpu.VMEM_SHARED((NACC, D), jnp.float32),
        ],
    )
    r3 = lambda a: a.astype(jnp.int32)
    r4 = lambda w: jnp.broadcast_to(w[:, None], (E, 16)).reshape(E * 16)
    return f(ha, hp, hv,
             r3(ei_a[0]), r3(ei_p[0]), r3(ei_v[0]),
             r3(ei_a[1]), r3(ei_p[1]), r3(ei_v[1]),
             r4(ew_a), r4(ew_p), r4(ew_v),
             zeros, cntrows)


def _post_body(pa, pp, pv, cc, xn, u, wl, bl, out):
    x = xn[...]
    deg = cc[...][0] + cc[...][1]

    def unpack(p, r):
        pv2 = p[...]
        return (pv2[0] + pv2[1]) / jnp.maximum(deg[:, r:r + 1], 1.0)

    aggr_a = unpack(pa, 0)
    aggr_p = unpack(pp, 1)
    aggr_v = unpack(pv, 2)

    uu = u[...]
    u1 = uu[:D, :]
    u2 = uu[D:, :]
    xu = jnp.dot(x, u2, preferred_element_type=jnp.float32)

    def score(aggr):
        z = jnp.dot(aggr, u1, preferred_element_type=jnp.float32) + xu
        return jnp.exp(jnp.where(z > 0, z, 0.01 * z))

    sa = score(aggr_a)
    sp = score(aggr_p)
    sv = score(aggr_v)
    inv = 1.0 / (sa + sp + sv)
    comb = (sa * aggr_a + sp * aggr_p + sv * aggr_v) * inv

    w = wl[...]
    w1 = w[:, :D]
    w2 = w[:, D:]
    pre = jnp.dot(x, w1.T, preferred_element_type=jnp.float32) \
        + jnp.dot(comb, w2.T, preferred_element_type=jnp.float32) + bl[...]
    pre = jnp.maximum(pre, 0.0)
    norm = jnp.sqrt(jnp.sum(pre * pre, axis=1, keepdims=True))
    out[...] = pre / jnp.maximum(norm, 1e-12)


def _post(pa, pp, pv, cc, x_node, u, W_lin, b_lin):
    pspec = pl.BlockSpec((2, RBLK, D), lambda i: (0, i, 0))
    xspec = pl.BlockSpec((RBLK, D), lambda i: (i, 0))
    return pl.pallas_call(
        _post_body,
        grid=(pl.cdiv(N, RBLK),),
        in_specs=[pspec, pspec, pspec, pspec, xspec,
                  pl.BlockSpec((2 * D, 1), lambda i: (0, 0)),
                  pl.BlockSpec((D, 2 * D), lambda i: (0, 0)),
                  pl.BlockSpec((1, D), lambda i: (0, 0))],
        out_specs=xspec,
        out_shape=jax.ShapeDtypeStruct((N, D), jnp.float32),
    )(pa, pp, pv, cc, x_node, u, W_lin, b_lin.reshape(1, D))


def kernel(x_a, x_p, x_v, edge_index_a, edge_index_p, edge_index_v, x_node,
           num_node, edge_weight_a, edge_weight_p, edge_weight_v,
           W_agg_a, b_agg_a, W_agg_p, b_agg_p, W_agg_v, b_agg_v,
           u, W_lin, b_lin):
    ha, hp, hv = _pre(x_a, x_p, x_v, W_agg_a, W_agg_p, W_agg_v,
                      b_agg_a, b_agg_p, b_agg_v)
    pa, pp, pv, cc = _sc_edge_phase(
        ha, hp, hv, edge_index_a, edge_index_p, edge_index_v,
        edge_weight_a, edge_weight_p, edge_weight_v)
    return _post(pa, pp, pv, cc, x_node, u, W_lin, b_lin)


# speedup vs baseline: 7.2225x; 2.1094x over previous
import functools

import jax
import jax.numpy as jnp
from jax import lax
from jax.experimental import pallas as pl
from jax.experimental.pallas import tpu as pltpu
from jax.experimental.pallas import tpu_sc as plsc

N = 10000
E = 320000
D = 128
DA = D + 16
NTILES = 32
EPT = E // NTILES
CH = 80
NCH = EPT // CH
NACC = 10240
RPT = NACC // 16
RBLK = 1024


def _pre_body(xa, xp, xv, wa, wp, wv, ba, bp, bv, ha, hp, hv):
    ha[...] = jnp.maximum(jnp.dot(xa[...], wa[...].T,
                                  preferred_element_type=jnp.float32) + ba[...], 0.0)
    hp[...] = jnp.maximum(jnp.dot(xp[...], wp[...].T,
                                  preferred_element_type=jnp.float32) + bp[...], 0.0)
    hv[...] = jnp.maximum(jnp.dot(xv[...], wv[...].T,
                                  preferred_element_type=jnp.float32) + bv[...], 0.0)


def _pre(x_a, x_p, x_v, W_a, W_p, W_v, b_a, b_p, b_v):
    xspec = pl.BlockSpec((RBLK, D), lambda i: (i, 0))
    wspec = pl.BlockSpec((D, D), lambda i: (0, 0))
    bspec = pl.BlockSpec((1, D), lambda i: (0, 0))
    return pl.pallas_call(
        _pre_body,
        grid=(pl.cdiv(N, RBLK),),
        in_specs=[xspec, xspec, xspec, wspec, wspec, wspec, bspec, bspec, bspec],
        out_specs=[xspec, xspec, xspec],
        out_shape=[jax.ShapeDtypeStruct((N, D), jnp.float32)] * 3,
    )(x_a, x_p, x_v, W_a, W_p, W_v,
      b_a.reshape(1, D), b_p.reshape(1, D), b_v.reshape(1, D))


EPT = E // NTILES
NCH = EPT // CH
K2 = (NCH - 1) // 2
assert 2 * K2 + 1 == NCH


def _sc_body(ha, hp, hv, srca, srcp, srcv, tgta, tgtp, tgtv,
             wea, wep, wev, zeros_hbm, cntrows_hbm,
             outa, outp, outv, cnt_out,
             src_c0, src_c1, tgt_v, w_all, grows0, grows1,
             semg0, semg1, semc0, semc1, semt, acc):
    c = lax.axis_index("c")
    s = lax.axis_index("s")
    wid = c * 16 + s
    row0 = pl.multiple_of(s * RPT, 8)
    base = wid * EPT

    grows = (grows0, grows1)
    src_c = (src_c0, src_c1)
    semg = (semg0, semg1)
    semc = (semc0, semc1)

    def zero_acc():
        pltpu.sync_copy(zeros_hbm, acc.at[pl.ds(row0, RPT), :])

    def writeout(dst_hbm):
        pltpu.sync_copy(acc.at[pl.ds(row0, RPT), :],
                        dst_hbm.at[c, pl.ds(row0, RPT), :])

    def do_data(h_hbm, src3, tgt3, we3, out_hbm):
        zero_acc()
        ct = pltpu.async_copy(tgt3.at[pl.ds(base, EPT)], tgt_v, semt)
        cw = pltpu.async_copy(we3.at[pl.ds(base, EPT)], w_all, semg0)
        ct.wait()
        cw.wait()
        plsc.subcore_barrier()

        def g_start(k, b):
            pltpu.async_copy(h_hbm.at[tgt_v.at[pl.ds(k * CH, CH)]],
                             grows[b], semg[b])
            pltpu.async_copy(src3.at[pl.ds(base + k * CH, CH)],
                             src_c[b], semc[b])

        def g_wait(k, b):
            pltpu.make_async_copy(h_hbm.at[tgt_v.at[pl.ds(k * CH, CH)]],
                                  grows[b], semg[b]).wait()
            pltpu.make_async_copy(src3.at[pl.ds(base + k * CH, CH)],
                                  src_c[b], semc[b]).wait()

        def mult(k, b):
            gb = grows[b]

            def group(g, _):
                w16 = w_all[pl.ds(k * CH + g * 16, 16)]
                e0 = g * 16
                for l in range(16):
                    ws = lax.gather(
                        w16, jnp.full((16, 1), l, jnp.int32),
                        lax.GatherDimensionNumbers(
                            offset_dims=(), collapsed_slice_dims=(0,),
                            start_index_map=(0,)),
                        slice_sizes=(1,),
                        mode=lax.GatherScatterMode.PROMISE_IN_BOUNDS)
                    e = e0 + l
                    for j in range(D // 16):
                        gb[e, pl.ds(j * 16, 16)] = \
                            gb[e, pl.ds(j * 16, 16)] * ws
                return _
            lax.fori_loop(0, CH // 16, group, None)

        def chunk_step(k, b):
            g_wait(k, b)
            mult(k, b)
            pltpu.sync_copy(grows[b], acc.at[src_c[b]], add=True)

        g_start(0, 0)

        def duo(k2, _):
            k = 2 * k2
            g_start(k + 1, 1)
            chunk_step(k, 0)
            g_start(k + 2, 0)
            chunk_step(k + 1, 1)
            return _
        lax.fori_loop(0, K2, duo, None)

        chunk_step(NCH - 1, 0)
        plsc.subcore_barrier()
        writeout(out_hbm)
        plsc.subcore_barrier()

    def do_counts():
        zero_acc()
        plsc.subcore_barrier()

        def c_start(src3, k, b):
            pltpu.async_copy(src3.at[pl.ds(base + k * CH, CH)],
                             src_c[b], semc[b])

        def c_wait(src3, k, b):
            pltpu.make_async_copy(src3.at[pl.ds(base + k * CH, CH)],
                                  src_c[b], semc[b]).wait()

        for r, src3 in enumerate((srca, srcp, srcv)):
            pltpu.sync_copy(cntrows_hbm.at[r], grows0)
            c_start(src3, 0, 0)

            def step(src3, k, b):
                c_wait(src3, k, b)
                pltpu.sync_copy(grows0, acc.at[src_c[b]], add=True)

            def duo(k2, _):
                k = 2 * k2
                c_start(src3, k + 1, 1)
                step(src3, k, 0)
                c_start(src3, k + 2, 0)
                step(src3, k + 1, 1)
                return _
            lax.fori_loop(0, K2, duo, None)
            step(src3, NCH - 1, 0)
        plsc.subcore_barrier()
        writeout(cnt_out)
        plsc.subcore_barrier()

    do_data(ha, srca, tgta, wea, outa)
    do_data(hp, srcp, tgtp, wep, outp)
    do_data(hv, srcv, tgtv, wev, outv)
    do_counts()


def _sc_edge_phase(ha, hp, hv, ei_a, ei_p, ei_v, ew_a, ew_p, ew_v):
    zeros = jnp.zeros((RPT, D), jnp.float32)
    lane = jnp.arange(D, dtype=jnp.int32)
    cntrows = jnp.stack([
        jnp.broadcast_to((lane == r).astype(jnp.float32), (CH, D))
        for r in range(3)])
    mesh = plsc.VectorSubcoreMesh(core_axis_name="c", subcore_axis_name="s")
    f = pl.kernel(
        _sc_body,
        out_type=[jax.ShapeDtypeStruct((2, NACC, D), jnp.float32)] * 4,
        mesh=mesh,
        scratch_types=[
            pltpu.VMEM((CH,), jnp.int32),
            pltpu.VMEM((CH,), jnp.int32),
            pltpu.VMEM((EPT,), jnp.int32),
            pltpu.VMEM((EPT,), jnp.float32),
            pltpu.VMEM((CH, D), jnp.float32),
            pltpu.VMEM((CH, D), jnp.float32),
        ] + [pltpu.SemaphoreType.DMA] * 5 + [
            pltpu.VMEM_SHARED((NACC, D), jnp.float32),
        ],
    )
    r3 = lambda a: a.astype(jnp.int32)
    return f(ha, hp, hv,
             r3(ei_a[0]), r3(ei_p[0]), r3(ei_v[0]),
             r3(ei_a[1]), r3(ei_p[1]), r3(ei_v[1]),
             ew_a, ew_p, ew_v,
             zeros, cntrows)


def _post_body(pa, pp, pv, cc, xn, u, wl, bl, out):
    x = xn[...]
    deg = cc[...][0] + cc[...][1]

    def unpack(p, r):
        pv2 = p[...]
        return (pv2[0] + pv2[1]) / jnp.maximum(deg[:, r:r + 1], 1.0)

    aggr_a = unpack(pa, 0)
    aggr_p = unpack(pp, 1)
    aggr_v = unpack(pv, 2)

    uu = u[...]
    u1 = uu[:D, :]
    u2 = uu[D:, :]
    xu = jnp.dot(x, u2, preferred_element_type=jnp.float32)

    def score(aggr):
        z = jnp.dot(aggr, u1, preferred_element_type=jnp.float32) + xu
        return jnp.exp(jnp.where(z > 0, z, 0.01 * z))

    sa = score(aggr_a)
    sp = score(aggr_p)
    sv = score(aggr_v)
    inv = 1.0 / (sa + sp + sv)
    comb = (sa * aggr_a + sp * aggr_p + sv * aggr_v) * inv

    w = wl[...]
    w1 = w[:, :D]
    w2 = w[:, D:]
    pre = jnp.dot(x, w1.T, preferred_element_type=jnp.float32) \
        + jnp.dot(comb, w2.T, preferred_element_type=jnp.float32) + bl[...]
    pre = jnp.maximum(pre, 0.0)
    norm = jnp.sqrt(jnp.sum(pre * pre, axis=1, keepdims=True))
    out[...] = pre / jnp.maximum(norm, 1e-12)


def _post(pa, pp, pv, cc, x_node, u, W_lin, b_lin):
    pspec = pl.BlockSpec((2, RBLK, D), lambda i: (0, i, 0))
    xspec = pl.BlockSpec((RBLK, D), lambda i: (i, 0))
    return pl.pallas_call(
        _post_body,
        grid=(pl.cdiv(N, RBLK),),
        in_specs=[pspec, pspec, pspec, pspec, xspec,
                  pl.BlockSpec((2 * D, 1), lambda i: (0, 0)),
                  pl.BlockSpec((D, 2 * D), lambda i: (0, 0)),
                  pl.BlockSpec((1, D), lambda i: (0, 0))],
        out_specs=xspec,
        out_shape=jax.ShapeDtypeStruct((N, D), jnp.float32),
    )(pa, pp, pv, cc, x_node, u, W_lin, b_lin.reshape(1, D))


def kernel(x_a, x_p, x_v, edge_index_a, edge_index_p, edge_index_v, x_node,
           num_node, edge_weight_a, edge_weight_p, edge_weight_v,
           W_agg_a, b_agg_a, W_agg_p, b_agg_p, W_agg_v, b_agg_v,
           u, W_lin, b_lin):
    ha, hp, hv = _pre(x_a, x_p, x_v, W_agg_a, W_agg_p, W_agg_v,
                      b_agg_a, b_agg_p, b_agg_v)
    pa, pp, pv, cc = _sc_edge_phase(
        ha, hp, hv, edge_index_a, edge_index_p, edge_index_v,
        edge_weight_a, edge_weight_p, edge_weight_v)
    return _post(pa, pp, pv, cc, x_node, u, W_lin, b_lin)
